# Initial kernel scaffold; baseline (speedup 1.0000x reference)
import jax
import jax.numpy as jnp
from jax.experimental import pallas as pl


def kernel(x, Wg, bg, W1, b1, W2, b2):
    raise NotImplementedError("write your pallas kernel here")


# speedup vs baseline: 1.4302x; 1.4302x over previous
import functools

import jax
import jax.numpy as jnp
from jax import lax
from jax.experimental import pallas as pl
from jax.experimental.pallas import tpu as pltpu

B, D, H, O, E, K = 4096, 1024, 2048, 1024, 8, 2
BM = 256
NB = (B * K) // BM + E
NS = NB * BM
GB = 512

_INTERPRET = False


def _gate_body(x_ref, wg_ref, bg_ref, w_ref, i_ref):
    logits = (
        jnp.dot(x_ref[...], wg_ref[...], preferred_element_type=jnp.float32)
        + bg_ref[...]
    )
    a1 = jnp.argmax(logits, axis=1)
    m1 = jnp.max(logits, axis=1)
    lane = lax.broadcasted_iota(jnp.int32, logits.shape, 1)
    masked = jnp.where(lane == a1[:, None].astype(jnp.int32), -jnp.inf, logits)
    a2 = jnp.argmax(masked, axis=1)
    m2 = jnp.max(masked, axis=1)
    w1 = 1.0 / (1.0 + jnp.exp(m2 - m1))
    w_ref[...] = jnp.stack([w1, 1.0 - w1], axis=1)
    i_ref[...] = jnp.stack([a1.astype(jnp.int32), a2.astype(jnp.int32)], axis=1)


def _gating(x, Wg, bg):
    wgp = jnp.zeros((D, 128), jnp.float32).at[:, :E].set(Wg)
    bgp = jnp.full((1, 128), -1e30, jnp.float32).at[0, :E].set(bg)
    return pl.pallas_call(
        _gate_body,
        grid=(B // GB,),
        in_specs=[
            pl.BlockSpec((GB, D), lambda i: (i, 0)),
            pl.BlockSpec((D, 128), lambda i: (0, 0)),
            pl.BlockSpec((1, 128), lambda i: (0, 0)),
        ],
        out_specs=[
            pl.BlockSpec((GB, K), lambda i: (i, 0)),
            pl.BlockSpec((GB, K), lambda i: (i, 0)),
        ],
        out_shape=[
            jax.ShapeDtypeStruct((B, K), jnp.float32),
            jax.ShapeDtypeStruct((B, K), jnp.int32),
        ],
        interpret=_INTERPRET,
    )(x, wgp, bgp)


def _route(iout, wout):
    ef = iout.reshape(-1)
    oh = (ef[:, None] == jnp.arange(E, dtype=jnp.int32)[None, :]).astype(jnp.int32)
    cum = jnp.cumsum(oh, axis=0)
    counts = cum[-1]
    rank = jnp.take_along_axis(cum, ef[:, None], axis=1)[:, 0] - 1
    nblk = (counts + BM - 1) // BM
    ends = jnp.cumsum(nblk)
    start = (ends - nblk) * BM
    slot = start[ef] + rank
    rows_token = jnp.zeros((NS,), jnp.int32).at[slot].set(
        jnp.arange(B * K, dtype=jnp.int32) // K)
    wslot = jnp.zeros((NS,), jnp.float32).at[slot].set(wout.reshape(-1))
    blk = jnp.arange(NB, dtype=jnp.int32)
    block_expert = jnp.minimum(
        jnp.searchsorted(ends, blk, side="right").astype(jnp.int32), E - 1)
    return slot, rows_token, wslot, block_expert


def _moe_body(be_ref, xs_ref, w1_ref, b1_ref, w2_ref, b2_ref, ws_ref, ys_ref):
    h = jnp.maximum(
        jnp.dot(xs_ref[...], w1_ref[0], preferred_element_type=jnp.float32)
        + b1_ref[0], 0.0)
    y = jnp.dot(h, w2_ref[0], preferred_element_type=jnp.float32) + b2_ref[0]
    ys_ref[...] = y * ws_ref[...]


def _grouped_mlp(xs, W1, b1, W2, b2, wslot, block_expert):
    grid_spec = pltpu.PrefetchScalarGridSpec(
        num_scalar_prefetch=1,
        grid=(NB,),
        in_specs=[
            pl.BlockSpec((BM, D), lambda i, be: (i, 0)),
            pl.BlockSpec((1, D, H), lambda i, be: (be[i], 0, 0)),
            pl.BlockSpec((1, 1, H), lambda i, be: (be[i], 0, 0)),
            pl.BlockSpec((1, H, O), lambda i, be: (be[i], 0, 0)),
            pl.BlockSpec((1, 1, O), lambda i, be: (be[i], 0, 0)),
            pl.BlockSpec((BM, 1), lambda i, be: (i, 0)),
        ],
        out_specs=pl.BlockSpec((BM, O), lambda i, be: (i, 0)),
    )
    return pl.pallas_call(
        _moe_body,
        grid_spec=grid_spec,
        out_shape=jax.ShapeDtypeStruct((NS, O), jnp.float32),
        compiler_params=pltpu.CompilerParams(
            dimension_semantics=("arbitrary",)),
        interpret=_INTERPRET,
    )(block_expert, xs, W1, b1[:, None, :], W2, b2[:, None, :], wslot[:, None])


def kernel(x, Wg, bg, W1, b1, W2, b2):
    wout, iout = _gating(x, Wg, bg)
    slot, rows_token, wslot, block_expert = _route(iout, wout)
    xs = x[rows_token]
    ys = _grouped_mlp(xs, W1, b1, W2, b2, wslot, block_expert)
    s0, s1 = slot[0::2], slot[1::2]
    return ys[s0] + ys[s1]
